# Initial kernel scaffold; baseline (speedup 1.0000x reference)
import jax
import jax.numpy as jnp
from jax.experimental import pallas as pl


def kernel(x, edge_index, batch, W1l, b1, W1r, W2l, b2, W2r, Wlin, blin):
    raise NotImplementedError("write your pallas kernel here")


# speedup vs baseline: 4.3251x; 4.3251x over previous
import functools

import jax
import jax.numpy as jnp
from jax import lax
from jax.experimental import pallas as pl
from jax.experimental.pallas import tpu as pltpu
from jax.experimental.pallas import tpu_sc as plsc

N = 10000
E = 320000
F_IN = 128
H = 64
C = 40

NC = 2
NS = 16
NW = NC * NS
CK = 128
EW = 10240
NCHUNK = EW // CK
E_PAD = NW * EW
N_PAD = 10240
RPT = N_PAD // NS
DEGW = 16

BR = 512


def _lin1_body(x_ref, w_ref, b_ref, l_ref, r_ref):
    y = jnp.dot(x_ref[...], w_ref[...], preferred_element_type=jnp.float32)
    l_ref[...] = y[:, :H]
    r_ref[...] = y[:, H:] + b_ref[...]


def _lin1(x, w, b):
    return pl.pallas_call(
        _lin1_body,
        grid=(pl.cdiv(N, BR),),
        in_specs=[
            pl.BlockSpec((BR, F_IN), lambda i: (i, 0)),
            pl.BlockSpec((F_IN, 2 * H), lambda i: (0, 0)),
            pl.BlockSpec((1, H), lambda i: (0, 0)),
        ],
        out_specs=[
            pl.BlockSpec((BR, H), lambda i: (i, 0)),
            pl.BlockSpec((BR, H), lambda i: (i, 0)),
        ],
        out_shape=[
            jax.ShapeDtypeStruct((N, H), jnp.float32),
            jax.ShapeDtypeStruct((N, H), jnp.float32),
        ],
    )(x, w, b)


def _mid_body(a0_ref, a1_ref, d0_ref, d1_ref, xr_ref, w_ref, b_ref,
              l_ref, r_ref):
    deg = jnp.maximum(d0_ref[:, :1] + d1_ref[:, :1], 1.0)
    h = jnp.maximum((a0_ref[...] + a1_ref[...]) / deg + xr_ref[...], 0.0)
    y = jnp.dot(h, w_ref[...], preferred_element_type=jnp.float32)
    l_ref[...] = y[:, :H]
    r_ref[...] = y[:, H:] + b_ref[...]


def _mid(a0, a1, d0, d1, xr, w, b):
    return pl.pallas_call(
        _mid_body,
        grid=(pl.cdiv(N, BR),),
        in_specs=[
            pl.BlockSpec((BR, H), lambda i: (i, 0)),
            pl.BlockSpec((BR, H), lambda i: (i, 0)),
            pl.BlockSpec((BR, DEGW), lambda i: (i, 0)),
            pl.BlockSpec((BR, DEGW), lambda i: (i, 0)),
            pl.BlockSpec((BR, H), lambda i: (i, 0)),
            pl.BlockSpec((H, 2 * H), lambda i: (0, 0)),
            pl.BlockSpec((1, H), lambda i: (0, 0)),
        ],
        out_specs=[
            pl.BlockSpec((BR, H), lambda i: (i, 0)),
            pl.BlockSpec((BR, H), lambda i: (i, 0)),
        ],
        out_shape=[
            jax.ShapeDtypeStruct((N, H), jnp.float32),
            jax.ShapeDtypeStruct((N, H), jnp.float32),
        ],
    )(a0, a1, d0, d1, xr, w, b)


def _fin_body(a0_ref, a1_ref, d0_ref, d1_ref, hr_ref, w_ref, b_ref, o_ref):
    deg = jnp.maximum(d0_ref[:, :1] + d1_ref[:, :1], 1.0)
    h = jnp.maximum((a0_ref[...] + a1_ref[...]) / deg + hr_ref[...], 0.0)
    o_ref[...] = jnp.dot(h, w_ref[...],
                         preferred_element_type=jnp.float32) + b_ref[...]


def _fin(a0, a1, d0, d1, hr, w, b):
    return pl.pallas_call(
        _fin_body,
        grid=(pl.cdiv(N, BR),),
        in_specs=[
            pl.BlockSpec((BR, H), lambda i: (i, 0)),
            pl.BlockSpec((BR, H), lambda i: (i, 0)),
            pl.BlockSpec((BR, DEGW), lambda i: (i, 0)),
            pl.BlockSpec((BR, DEGW), lambda i: (i, 0)),
            pl.BlockSpec((BR, H), lambda i: (i, 0)),
            pl.BlockSpec((H, C), lambda i: (0, 0)),
            pl.BlockSpec((1, C), lambda i: (0, 0)),
        ],
        out_specs=pl.BlockSpec((BR, C), lambda i: (i, 0)),
        out_shape=jax.ShapeDtypeStruct((N, C), jnp.float32),
    )(a0, a1, d0, d1, hr, w, b)


def _agg_deg_body(table, src_h, dst_h, zrow_h, zdeg_h, acc_out, deg_out,
                  src_v, dst_v, rows_v, ones_v, acc_s, deg_s, sem):
    cid = lax.axis_index("c")
    sid = lax.axis_index("s")
    wid = sid * NC + cid
    pltpu.sync_copy(src_h.at[wid], src_v)
    pltpu.sync_copy(dst_h.at[wid], dst_v)
    r0 = sid * RPT
    pltpu.sync_copy(zrow_h, acc_s.at[pl.ds(r0, RPT)])
    pltpu.sync_copy(zdeg_h, deg_s.at[pl.ds(r0, RPT)])
    one = jnp.ones((DEGW,), jnp.float32)
    for r in range(CK):
        ones_v[r, :] = one
    plsc.subcore_barrier()

    def step(j, carry):
        pltpu.async_copy(table.at[src_v.at[j]], rows_v, sem).wait()
        pltpu.sync_copy(rows_v, acc_s.at[dst_v.at[j]], add=True)
        pltpu.sync_copy(ones_v, deg_s.at[dst_v.at[j]], add=True)
        return carry

    lax.fori_loop(0, NCHUNK, step, 0)
    plsc.subcore_barrier()
    pltpu.sync_copy(acc_s.at[pl.ds(r0, RPT)], acc_out.at[cid, pl.ds(r0, RPT)])
    pltpu.sync_copy(deg_s.at[pl.ds(r0, RPT)], deg_out.at[cid, pl.ds(r0, RPT)])


def _agg_body(table, src_h, dst_h, zrow_h, acc_out,
              src_v, dst_v, rows_v, acc_s, sem):
    cid = lax.axis_index("c")
    sid = lax.axis_index("s")
    wid = sid * NC + cid
    pltpu.sync_copy(src_h.at[wid], src_v)
    pltpu.sync_copy(dst_h.at[wid], dst_v)
    r0 = sid * RPT
    pltpu.sync_copy(zrow_h, acc_s.at[pl.ds(r0, RPT)])
    plsc.subcore_barrier()

    def step(j, carry):
        pltpu.async_copy(table.at[src_v.at[j]], rows_v, sem).wait()
        pltpu.sync_copy(rows_v, acc_s.at[dst_v.at[j]], add=True)
        return carry

    lax.fori_loop(0, NCHUNK, step, 0)
    plsc.subcore_barrier()
    pltpu.sync_copy(acc_s.at[pl.ds(r0, RPT)], acc_out.at[cid, pl.ds(r0, RPT)])


_SC_MESH = dict(core_axis_name="c", subcore_axis_name="s")


def _agg_deg(table, src, dst, zrow, zdeg):
    return pl.kernel(
        _agg_deg_body,
        out_type=(
            jax.ShapeDtypeStruct((NC, N_PAD, H), jnp.float32),
            jax.ShapeDtypeStruct((NC, N_PAD, DEGW), jnp.float32),
        ),
        mesh=plsc.VectorSubcoreMesh(**_SC_MESH),
        compiler_params=pltpu.CompilerParams(use_tc_tiling_on_sc=False),
        scratch_types=[
            pltpu.VMEM((NCHUNK, CK), jnp.int32),
            pltpu.VMEM((NCHUNK, CK), jnp.int32),
            pltpu.VMEM((CK, H), jnp.float32),
            pltpu.VMEM((CK, DEGW), jnp.float32),
            pltpu.VMEM_SHARED((N_PAD, H), jnp.float32),
            pltpu.VMEM_SHARED((N_PAD, DEGW), jnp.float32),
            pltpu.SemaphoreType.DMA,
        ],
    )(table, src, dst, zrow, zdeg)


def _agg(table, src, dst, zrow):
    return pl.kernel(
        _agg_body,
        out_type=jax.ShapeDtypeStruct((NC, N_PAD, H), jnp.float32),
        mesh=plsc.VectorSubcoreMesh(**_SC_MESH),
        compiler_params=pltpu.CompilerParams(use_tc_tiling_on_sc=False),
        scratch_types=[
            pltpu.VMEM((NCHUNK, CK), jnp.int32),
            pltpu.VMEM((NCHUNK, CK), jnp.int32),
            pltpu.VMEM((CK, H), jnp.float32),
            pltpu.VMEM_SHARED((N_PAD, H), jnp.float32),
            pltpu.SemaphoreType.DMA,
        ],
    )(table, src, dst, zrow)


def kernel(x, edge_index, batch, W1l, b1, W1r, W2l, b2, W2r, Wlin, blin):
    src = edge_index[0]
    dst = edge_index[1]
    pad = E_PAD - E
    srcp = jnp.concatenate(
        [src, jnp.zeros((pad,), jnp.int32)]).reshape(NW, NCHUNK, CK)
    dstp = jnp.concatenate(
        [dst, jnp.full((pad,), N, jnp.int32)]).reshape(NW, NCHUNK, CK)
    zrow = jnp.zeros((RPT, H), jnp.float32)
    zdeg = jnp.zeros((RPT, DEGW), jnp.float32)

    W1 = jnp.concatenate([W1l, W1r], axis=1)
    W2 = jnp.concatenate([W2l, W2r], axis=1)

    xl, xr1 = _lin1(x, W1, b1.reshape(1, H))
    acc1, deg1 = _agg_deg(xl, srcp, dstp, zrow, zdeg)
    hl, hr2 = _mid(acc1[0, :N], acc1[1, :N], deg1[0, :N], deg1[1, :N],
                   xr1, W2, b2.reshape(1, H))
    acc2 = _agg(hl, srcp, dstp, zrow)
    return _fin(acc2[0, :N], acc2[1, :N], deg1[0, :N], deg1[1, :N],
                hr2, Wlin, blin.reshape(1, C))


# speedup vs baseline: 4.9928x; 1.1544x over previous
import functools

import jax
import jax.numpy as jnp
from jax import lax
from jax.experimental import pallas as pl
from jax.experimental.pallas import tpu as pltpu
from jax.experimental.pallas import tpu_sc as plsc

N = 10000
E = 320000
F_IN = 128
H = 64
C = 40

NC = 2
NS = 16
NW = NC * NS
CK = 128
EW = 10240
NCHUNK = EW // CK
E_PAD = NW * EW
N_PAD = 10240
RPT = N_PAD // NS
DEGW = 16

BR = 512


def _lin1_body(x_ref, w_ref, b_ref, l_ref, r_ref):
    y = jnp.dot(x_ref[...], w_ref[...], preferred_element_type=jnp.float32)
    l_ref[...] = y[:, :H]
    r_ref[...] = y[:, H:] + b_ref[...]


def _lin1(x, w, b):
    return pl.pallas_call(
        _lin1_body,
        grid=(pl.cdiv(N, BR),),
        in_specs=[
            pl.BlockSpec((BR, F_IN), lambda i: (i, 0)),
            pl.BlockSpec((F_IN, 2 * H), lambda i: (0, 0)),
            pl.BlockSpec((1, H), lambda i: (0, 0)),
        ],
        out_specs=[
            pl.BlockSpec((BR, H), lambda i: (i, 0)),
            pl.BlockSpec((BR, H), lambda i: (i, 0)),
        ],
        out_shape=[
            jax.ShapeDtypeStruct((N, H), jnp.float32),
            jax.ShapeDtypeStruct((N, H), jnp.float32),
        ],
    )(x, w, b)


def _mid_body(a0_ref, a1_ref, d0_ref, d1_ref, xr_ref, w_ref, b_ref,
              l_ref, r_ref):
    deg = jnp.maximum(d0_ref[:, :1] + d1_ref[:, :1], 1.0)
    h = jnp.maximum((a0_ref[...] + a1_ref[...]) / deg + xr_ref[...], 0.0)
    y = jnp.dot(h, w_ref[...], preferred_element_type=jnp.float32)
    l_ref[...] = y[:, :H]
    r_ref[...] = y[:, H:] + b_ref[...]


def _mid(a0, a1, d0, d1, xr, w, b):
    return pl.pallas_call(
        _mid_body,
        grid=(pl.cdiv(N, BR),),
        in_specs=[
            pl.BlockSpec((BR, H), lambda i: (i, 0)),
            pl.BlockSpec((BR, H), lambda i: (i, 0)),
            pl.BlockSpec((BR, DEGW), lambda i: (i, 0)),
            pl.BlockSpec((BR, DEGW), lambda i: (i, 0)),
            pl.BlockSpec((BR, H), lambda i: (i, 0)),
            pl.BlockSpec((H, 2 * H), lambda i: (0, 0)),
            pl.BlockSpec((1, H), lambda i: (0, 0)),
        ],
        out_specs=[
            pl.BlockSpec((BR, H), lambda i: (i, 0)),
            pl.BlockSpec((BR, H), lambda i: (i, 0)),
        ],
        out_shape=[
            jax.ShapeDtypeStruct((N, H), jnp.float32),
            jax.ShapeDtypeStruct((N, H), jnp.float32),
        ],
    )(a0, a1, d0, d1, xr, w, b)


def _fin_body(a0_ref, a1_ref, d0_ref, d1_ref, hr_ref, w_ref, b_ref, o_ref):
    deg = jnp.maximum(d0_ref[:, :1] + d1_ref[:, :1], 1.0)
    h = jnp.maximum((a0_ref[...] + a1_ref[...]) / deg + hr_ref[...], 0.0)
    o_ref[...] = jnp.dot(h, w_ref[...],
                         preferred_element_type=jnp.float32) + b_ref[...]


def _fin(a0, a1, d0, d1, hr, w, b):
    return pl.pallas_call(
        _fin_body,
        grid=(pl.cdiv(N, BR),),
        in_specs=[
            pl.BlockSpec((BR, H), lambda i: (i, 0)),
            pl.BlockSpec((BR, H), lambda i: (i, 0)),
            pl.BlockSpec((BR, DEGW), lambda i: (i, 0)),
            pl.BlockSpec((BR, DEGW), lambda i: (i, 0)),
            pl.BlockSpec((BR, H), lambda i: (i, 0)),
            pl.BlockSpec((H, C), lambda i: (0, 0)),
            pl.BlockSpec((1, C), lambda i: (0, 0)),
        ],
        out_specs=pl.BlockSpec((BR, C), lambda i: (i, 0)),
        out_shape=jax.ShapeDtypeStruct((N, C), jnp.float32),
    )(a0, a1, d0, d1, hr, w, b)


NBUF = 4


def _agg_deg_body(table, src_h, dst_h, zrow_h, zdeg_h, acc_out, deg_out,
                  src_v, dst_v, rows_v, ones_v, acc_s, deg_s, gsem, dsem):
    cid = lax.axis_index("c")
    sid = lax.axis_index("s")
    wid = sid * NC + cid
    pltpu.sync_copy(src_h.at[wid], src_v)
    pltpu.sync_copy(dst_h.at[wid], dst_v)
    r0 = sid * RPT
    pltpu.sync_copy(zrow_h, acc_s.at[pl.ds(r0, RPT)])
    pltpu.sync_copy(zdeg_h, deg_s.at[pl.ds(r0, RPT)])
    one = jnp.ones((DEGW,), jnp.float32)
    for r in range(CK):
        ones_v[r, :] = one
    plsc.subcore_barrier()

    def gstart(j, b):
        pltpu.async_copy(table.at[src_v.at[j]], rows_v.at[b], gsem.at[b])

    def gwait(j, b):
        pltpu.make_async_copy(
            table.at[src_v.at[j]], rows_v.at[b], gsem.at[b]).wait()

    def drain(j, b):
        gwait(j, b)
        pltpu.sync_copy(rows_v.at[b], acc_s.at[dst_v.at[j]], add=True)
        pltpu.make_async_copy(ones_v, deg_s.at[dst_v.at[j]], dsem).wait()

    for b in range(NBUF - 1):
        gstart(b, b)

    def step(i, carry):
        base = i * NBUF
        for b in range(NBUF):
            j = base + b
            gstart(j + NBUF - 1, (b + NBUF - 1) % NBUF)
            pltpu.async_copy(ones_v, deg_s.at[dst_v.at[j]], dsem, add=True)
            drain(j, b)
        return carry

    lax.fori_loop(0, NCHUNK // NBUF - 1, step, 0)
    base = NCHUNK - NBUF
    gstart(NCHUNK - 1, NBUF - 1)
    for b in range(NBUF):
        j = base + b
        pltpu.async_copy(ones_v, deg_s.at[dst_v.at[j]], dsem, add=True)
        drain(j, b)
    plsc.subcore_barrier()
    pltpu.sync_copy(acc_s.at[pl.ds(r0, RPT)], acc_out.at[cid, pl.ds(r0, RPT)])
    pltpu.sync_copy(deg_s.at[pl.ds(r0, RPT)], deg_out.at[cid, pl.ds(r0, RPT)])


def _agg_body(table, src_h, dst_h, zrow_h, acc_out,
              src_v, dst_v, rows_v, acc_s, gsem):
    cid = lax.axis_index("c")
    sid = lax.axis_index("s")
    wid = sid * NC + cid
    pltpu.sync_copy(src_h.at[wid], src_v)
    pltpu.sync_copy(dst_h.at[wid], dst_v)
    r0 = sid * RPT
    pltpu.sync_copy(zrow_h, acc_s.at[pl.ds(r0, RPT)])
    plsc.subcore_barrier()

    def gstart(j, b):
        pltpu.async_copy(table.at[src_v.at[j]], rows_v.at[b], gsem.at[b])

    def drain(j, b):
        pltpu.make_async_copy(
            table.at[src_v.at[j]], rows_v.at[b], gsem.at[b]).wait()
        pltpu.sync_copy(rows_v.at[b], acc_s.at[dst_v.at[j]], add=True)

    for b in range(NBUF - 1):
        gstart(b, b)

    def step(i, carry):
        base = i * NBUF
        for b in range(NBUF):
            j = base + b
            gstart(j + NBUF - 1, (b + NBUF - 1) % NBUF)
            drain(j, b)
        return carry

    lax.fori_loop(0, NCHUNK // NBUF - 1, step, 0)
    base = NCHUNK - NBUF
    gstart(NCHUNK - 1, NBUF - 1)
    for b in range(NBUF):
        drain(base + b, b)
    plsc.subcore_barrier()
    pltpu.sync_copy(acc_s.at[pl.ds(r0, RPT)], acc_out.at[cid, pl.ds(r0, RPT)])


_SC_MESH = dict(core_axis_name="c", subcore_axis_name="s")


def _agg_deg(table, src, dst, zrow, zdeg):
    return pl.kernel(
        _agg_deg_body,
        out_type=(
            jax.ShapeDtypeStruct((NC, N_PAD, H), jnp.float32),
            jax.ShapeDtypeStruct((NC, N_PAD, DEGW), jnp.float32),
        ),
        mesh=plsc.VectorSubcoreMesh(**_SC_MESH),
        compiler_params=pltpu.CompilerParams(use_tc_tiling_on_sc=False),
        scratch_types=[
            pltpu.VMEM((NCHUNK, CK), jnp.int32),
            pltpu.VMEM((NCHUNK, CK), jnp.int32),
            pltpu.VMEM((NBUF, CK, H), jnp.float32),
            pltpu.VMEM((CK, DEGW), jnp.float32),
            pltpu.VMEM_SHARED((N_PAD, H), jnp.float32),
            pltpu.VMEM_SHARED((N_PAD, DEGW), jnp.float32),
            pltpu.SemaphoreType.DMA((NBUF,)),
            pltpu.SemaphoreType.DMA,
        ],
    )(table, src, dst, zrow, zdeg)


def _agg(table, src, dst, zrow):
    return pl.kernel(
        _agg_body,
        out_type=jax.ShapeDtypeStruct((NC, N_PAD, H), jnp.float32),
        mesh=plsc.VectorSubcoreMesh(**_SC_MESH),
        compiler_params=pltpu.CompilerParams(use_tc_tiling_on_sc=False),
        scratch_types=[
            pltpu.VMEM((NCHUNK, CK), jnp.int32),
            pltpu.VMEM((NCHUNK, CK), jnp.int32),
            pltpu.VMEM((NBUF, CK, H), jnp.float32),
            pltpu.VMEM_SHARED((N_PAD, H), jnp.float32),
            pltpu.SemaphoreType.DMA((NBUF,)),
        ],
    )(table, src, dst, zrow)


def kernel(x, edge_index, batch, W1l, b1, W1r, W2l, b2, W2r, Wlin, blin):
    src = edge_index[0]
    dst = edge_index[1]
    pad = E_PAD - E
    srcp = jnp.concatenate(
        [src, jnp.zeros((pad,), jnp.int32)]).reshape(NW, NCHUNK, CK)
    dstp = jnp.concatenate(
        [dst, jnp.full((pad,), N, jnp.int32)]).reshape(NW, NCHUNK, CK)
    zrow = jnp.zeros((RPT, H), jnp.float32)
    zdeg = jnp.zeros((RPT, DEGW), jnp.float32)

    W1 = jnp.concatenate([W1l, W1r], axis=1)
    W2 = jnp.concatenate([W2l, W2r], axis=1)

    xl, xr1 = _lin1(x, W1, b1.reshape(1, H))
    acc1, deg1 = _agg_deg(xl, srcp, dstp, zrow, zdeg)
    hl, hr2 = _mid(acc1[0, :N], acc1[1, :N], deg1[0, :N], deg1[1, :N],
                   xr1, W2, b2.reshape(1, H))
    acc2 = _agg(hl, srcp, dstp, zrow)
    return _fin(acc2[0, :N], acc2[1, :N], deg1[0, :N], deg1[1, :N],
                hr2, Wlin, blin.reshape(1, C))
